# Initial kernel scaffold; baseline (speedup 1.0000x reference)
import jax
import jax.numpy as jnp
from jax.experimental import pallas as pl


def kernel(nodal_values, nodes, elements, quad_points, quad_weights):
    raise NotImplementedError("write your pallas kernel here")


# speedup vs baseline: 75.0502x; 75.0502x over previous
import functools

import jax
import jax.numpy as jnp
from jax import lax
from jax.experimental import pallas as pl
from jax.experimental.pallas import tpu as pltpu
from jax.experimental.pallas import tpu_sc as plsc

_NC = 2
_NS = 16
_NW = _NC * _NS
_L = 16

_EG = 16
_RG = 3 * _EG


def _sc_body(nvals_hbm, nodes_hbm, idx_hbm, coef_hbm, out_hbm,
             nodes_v, idxb, rows_v, wbuf, coefs_v, accv, sem):
    n_groups = idx_hbm.shape[0] // _RG
    wid = lax.axis_index("s") * _NC + lax.axis_index("c")
    g_start = wid * n_groups // _NW
    g_end = (wid + 1) * n_groups // _NW

    pltpu.sync_copy(nodes_hbm, nodes_v)
    pltpu.sync_copy(coef_hbm, coefs_v)
    cv = coefs_v[...]
    m00 = cv[0]
    m11 = cv[1]
    m22 = cv[2]
    m01 = cv[3]
    m02 = cv[4]
    m12 = cv[5]
    ws = cv[6]

    lane = lax.iota(jnp.int32, _L)
    lane3 = lane * 3
    zeros = jnp.zeros((_L,), jnp.int32)
    ones = jnp.ones((_L,), jnp.int32)

    def group_body(g, acc):
        pltpu.sync_copy(idx_hbm.at[pl.ds(g * _RG, _RG)], idxb)
        gather = pltpu.async_copy(nvals_hbm.at[idxb], rows_v, sem)

        ids0 = plsc.load_gather(idxb, [lane3]) * 2
        ids1 = plsc.load_gather(idxb, [lane3 + 1]) * 2
        ids2 = plsc.load_gather(idxb, [lane3 + 2]) * 2
        x0 = plsc.load_gather(nodes_v, [ids0])
        y0 = plsc.load_gather(nodes_v, [ids0 + 1])
        x1 = plsc.load_gather(nodes_v, [ids1])
        y1 = plsc.load_gather(nodes_v, [ids1 + 1])
        x2 = plsc.load_gather(nodes_v, [ids2])
        y2 = plsc.load_gather(nodes_v, [ids2 + 1])
        e1x = x1 - x0
        e1y = y1 - y0
        e2x = x2 - x0
        e2y = y2 - y0
        detj = e1x * e2y - e1y * e2x
        ag = e1x * e1x + e1y * e1y
        bg = e1x * e2x + e1y * e2y
        cg = e2x * e2x + e2y * e2y
        s = ws / detj
        wbuf[pl.ds(0, _L)] = detj * m00 + s * (ag - 2.0 * bg + cg)
        wbuf[pl.ds(_L, _L)] = detj * m11 + s * cg
        wbuf[pl.ds(2 * _L, _L)] = detj * m22 + s * ag
        wbuf[pl.ds(3 * _L, _L)] = 2.0 * (detj * m01 - s * (cg - bg))
        wbuf[pl.ds(4 * _L, _L)] = 2.0 * (detj * m02 - s * (ag - bg))
        wbuf[pl.ds(5 * _L, _L)] = 2.0 * (detj * m12 - s * bg)

        gather.wait()

        def el_body(l, a_in):
            li = jnp.full((_L,), l, dtype=jnp.int32)
            w0 = plsc.load_gather(wbuf, [li])
            w1 = plsc.load_gather(wbuf, [li + _L])
            w2 = plsc.load_gather(wbuf, [li + 2 * _L])
            w3 = plsc.load_gather(wbuf, [li + 3 * _L])
            w4 = plsc.load_gather(wbuf, [li + 4 * _L])
            w5 = plsc.load_gather(wbuf, [li + 5 * _L])
            r0 = l * 3
            a_out = a_in
            for j in range(0, 256, _L):
                va = rows_v[r0, pl.ds(j, _L)]
                vb = rows_v[r0 + 1, pl.ds(j, _L)]
                vc = rows_v[r0 + 2, pl.ds(j, _L)]
                a_out = a_out + ((w0 * va + w3 * vb + w4 * vc) * va
                                 + (w1 * vb + w5 * vc) * vb
                                 + (w2 * vc) * vc)
            return a_out

        return lax.fori_loop(0, _EG, el_body, acc)

    acc = lax.fori_loop(g_start, g_end, group_body,
                        jnp.zeros((_L,), jnp.float32))
    accv[...] = acc
    pltpu.sync_copy(accv, out_hbm.at[wid])


def _run_sc(nvals, nodes_xy, idx_flat, coefs):
    n_nodes = nodes_xy.shape[0] // 2
    mesh = plsc.VectorSubcoreMesh(core_axis_name="c", subcore_axis_name="s")
    f = pl.kernel(
        _sc_body,
        out_type=jax.ShapeDtypeStruct((_NW, _L), jnp.float32),
        mesh=mesh,
        scratch_types=[
            pltpu.VMEM((2 * n_nodes,), jnp.float32),
            pltpu.VMEM((_RG,), jnp.int32),
            pltpu.VMEM((_RG, 256), jnp.float32),
            pltpu.VMEM((6 * _L,), jnp.float32),
            pltpu.VMEM((_L,), jnp.float32),
            pltpu.VMEM((_L,), jnp.float32),
            pltpu.SemaphoreType.DMA,
        ],
        compiler_params=pltpu.CompilerParams(needs_layout_passes=False),
    )
    return f(nvals, nodes_xy, idx_flat, coefs)


def kernel(nodal_values, nodes, elements, quad_points, quad_weights):
    qx = quad_points[:, 0]
    qy = quad_points[:, 1]
    shp = jnp.stack([1.0 - qx - qy, qx, qy], axis=1)
    m = jnp.einsum('q,qa,qb->ab', quad_weights, shp, shp)
    coefs = jnp.zeros((16,), jnp.float32)
    coefs = coefs.at[:7].set(jnp.stack(
        [m[0, 0], m[1, 1], m[2, 2], m[0, 1], m[0, 2], m[1, 2],
         jnp.sum(quad_weights)]))
    idx_flat = elements.reshape(-1).astype(jnp.int32)
    partials = _run_sc(nodal_values, nodes.reshape(-1), idx_flat, coefs)
    return jnp.sum(partials)


# speedup vs baseline: 101.0945x; 1.3470x over previous
import functools

import jax
import jax.numpy as jnp
from jax import lax
from jax.experimental import pallas as pl
from jax.experimental.pallas import tpu as pltpu
from jax.experimental.pallas import tpu_sc as plsc

_NC = 2
_NS = 16
_NW = _NC * _NS
_L = 16

_EPG = 32
_RPG = 3 * _EPG
_SUB = _EPG // _L


def _sc_body(nvals_hbm, nodes_hbm, idx_hbm, coef_hbm, out_hbm,
             nodes_v, idxb0, idxb1, rows0, rows1, wbuf, coefs_v, accv,
             sem0, sem1):
    n_units = idx_hbm.shape[0] // (2 * _RPG)
    wid = lax.axis_index("s") * _NC + lax.axis_index("c")
    u_start = wid * n_units // _NW
    u_end = (wid + 1) * n_units // _NW
    g_last = 2 * u_end - 1

    pltpu.sync_copy(nodes_hbm, nodes_v)
    pltpu.sync_copy(coef_hbm, coefs_v)
    cv = coefs_v[...]
    m00 = cv[0]
    m11 = cv[1]
    m22 = cv[2]
    m01 = cv[3]
    m02 = cv[4]
    m12 = cv[5]
    ws = cv[6]

    lane = lax.iota(jnp.int32, _L)
    lane3 = lane * 3

    def issue(g, idxb, rowsb, sem):
        pltpu.sync_copy(idx_hbm.at[pl.ds(g * _RPG, _RPG)], idxb)
        pltpu.async_copy(nvals_hbm.at[idxb], rowsb, sem)

    def drain(idxb, rowsb, sem):
        pltpu.make_async_copy(nvals_hbm.at[idxb], rowsb, sem).wait()

    def compute(idxb, rowsb, acc):
        for s in range(_SUB):
            ids0 = plsc.load_gather(idxb, [lane3 + 48 * s]) * 2
            ids1 = plsc.load_gather(idxb, [lane3 + 48 * s + 1]) * 2
            ids2 = plsc.load_gather(idxb, [lane3 + 48 * s + 2]) * 2
            x0 = plsc.load_gather(nodes_v, [ids0])
            y0 = plsc.load_gather(nodes_v, [ids0 + 1])
            x1 = plsc.load_gather(nodes_v, [ids1])
            y1 = plsc.load_gather(nodes_v, [ids1 + 1])
            x2 = plsc.load_gather(nodes_v, [ids2])
            y2 = plsc.load_gather(nodes_v, [ids2 + 1])
            e1x = x1 - x0
            e1y = y1 - y0
            e2x = x2 - x0
            e2y = y2 - y0
            detj = e1x * e2y - e1y * e2x
            ag = e1x * e1x + e1y * e1y
            bg = e1x * e2x + e1y * e2y
            cg = e2x * e2x + e2y * e2y
            sc = ws / detj
            wbuf[pl.ds(0, _L)] = detj * m00 + sc * (ag - 2.0 * bg + cg)
            wbuf[pl.ds(_L, _L)] = detj * m11 + sc * cg
            wbuf[pl.ds(2 * _L, _L)] = detj * m22 + sc * ag
            wbuf[pl.ds(3 * _L, _L)] = 2.0 * (detj * m01 - sc * (cg - bg))
            wbuf[pl.ds(4 * _L, _L)] = 2.0 * (detj * m02 - sc * (ag - bg))
            wbuf[pl.ds(5 * _L, _L)] = 2.0 * (detj * m12 - sc * bg)

            def el_body(l, a_in):
                li = jnp.full((_L,), l, dtype=jnp.int32)
                w0 = plsc.load_gather(wbuf, [li])
                w1 = plsc.load_gather(wbuf, [li + _L])
                w2 = plsc.load_gather(wbuf, [li + 2 * _L])
                w3 = plsc.load_gather(wbuf, [li + 3 * _L])
                w4 = plsc.load_gather(wbuf, [li + 4 * _L])
                w5 = plsc.load_gather(wbuf, [li + 5 * _L])
                r0 = 48 * s + l * 3
                a_out = a_in
                for j in range(0, 256, _L):
                    va = rowsb[r0, pl.ds(j, _L)]
                    vb = rowsb[r0 + 1, pl.ds(j, _L)]
                    vc = rowsb[r0 + 2, pl.ds(j, _L)]
                    a_out = a_out + ((w0 * va + w3 * vb + w4 * vc) * va
                                     + (w1 * vb + w5 * vc) * vb
                                     + (w2 * vc) * vc)
                return a_out

            acc = lax.fori_loop(0, _EG_INNER, el_body, acc)
        return acc

    issue(2 * u_start, idxb0, rows0, sem0)

    def unit_body(u, acc):
        issue(2 * u + 1, idxb1, rows1, sem1)
        drain(idxb0, rows0, sem0)
        acc = compute(idxb0, rows0, acc)
        issue(jnp.minimum(2 * u + 2, g_last), idxb0, rows0, sem0)
        drain(idxb1, rows1, sem1)
        acc = compute(idxb1, rows1, acc)
        return acc

    acc = lax.fori_loop(u_start, u_end, unit_body,
                        jnp.zeros((_L,), jnp.float32))
    drain(idxb0, rows0, sem0)
    accv[...] = acc
    pltpu.sync_copy(accv, out_hbm.at[wid])


_EG_INNER = 16


def _run_sc(nvals, nodes_xy, idx_flat, coefs):
    n_nodes = nodes_xy.shape[0] // 2
    mesh = plsc.VectorSubcoreMesh(core_axis_name="c", subcore_axis_name="s")
    f = pl.kernel(
        _sc_body,
        out_type=jax.ShapeDtypeStruct((_NW, _L), jnp.float32),
        mesh=mesh,
        scratch_types=[
            pltpu.VMEM((2 * n_nodes,), jnp.float32),
            pltpu.VMEM((_RPG,), jnp.int32),
            pltpu.VMEM((_RPG,), jnp.int32),
            pltpu.VMEM((_RPG, 256), jnp.float32),
            pltpu.VMEM((_RPG, 256), jnp.float32),
            pltpu.VMEM((6 * _L,), jnp.float32),
            pltpu.VMEM((_L,), jnp.float32),
            pltpu.VMEM((_L,), jnp.float32),
            pltpu.SemaphoreType.DMA,
            pltpu.SemaphoreType.DMA,
        ],
        compiler_params=pltpu.CompilerParams(needs_layout_passes=False),
    )
    return f(nvals, nodes_xy, idx_flat, coefs)


def kernel(nodal_values, nodes, elements, quad_points, quad_weights):
    qx = quad_points[:, 0]
    qy = quad_points[:, 1]
    shp = jnp.stack([1.0 - qx - qy, qx, qy], axis=1)
    m = jnp.einsum('q,qa,qb->ab', quad_weights, shp, shp)
    coefs = jnp.zeros((16,), jnp.float32)
    coefs = coefs.at[:7].set(jnp.stack(
        [m[0, 0], m[1, 1], m[2, 2], m[0, 1], m[0, 2], m[1, 2],
         jnp.sum(quad_weights)]))
    idx_flat = elements.reshape(-1).astype(jnp.int32)
    partials = _run_sc(nodal_values, nodes.reshape(-1), idx_flat, coefs)
    return jnp.sum(partials)


# speedup vs baseline: 155.5015x; 1.5382x over previous
import functools

import jax
import jax.numpy as jnp
from jax import lax
from jax.experimental import pallas as pl
from jax.experimental.pallas import tpu as pltpu
from jax.experimental.pallas import tpu_sc as plsc

_NC = 2
_NS = 16
_NW = _NC * _NS
_L = 16

_EPG = 32
_RPG = 3 * _EPG
_SUB = _EPG // _L


def _sc_body(nvals_hbm, nodes_hbm, idx_hbm, coef_hbm, out_hbm,
             nodes_v, idxb0, idxb1, rows0, rows1, wbuf, coefs_v, accv,
             sem0, sem1):
    n_units = idx_hbm.shape[0] // (2 * _RPG)
    wid = lax.axis_index("s") * _NC + lax.axis_index("c")
    u_start = wid * n_units // _NW
    u_end = (wid + 1) * n_units // _NW
    g_last = 2 * u_end - 1

    pltpu.sync_copy(nodes_hbm, nodes_v)
    pltpu.sync_copy(coef_hbm, coefs_v)
    cv = coefs_v[...]
    m00 = cv[0]
    m11 = cv[1]
    m22 = cv[2]
    m01 = cv[3]
    m02 = cv[4]
    m12 = cv[5]
    ws = cv[6]

    lane = lax.iota(jnp.int32, _L)
    lane3 = lane * 3

    def issue(g, idxb, rowsb, sem):
        pltpu.sync_copy(idx_hbm.at[pl.ds(g * _RPG, _RPG)], idxb)
        pltpu.async_copy(nvals_hbm.at[idxb], rowsb, sem)

    def drain(idxb, rowsb, sem):
        pltpu.make_async_copy(nvals_hbm.at[idxb], rowsb, sem).wait()

    def compute(idxb, rowsb, acc):
        for s in range(_SUB):
            ids0 = plsc.load_gather(idxb, [lane3 + 48 * s]) * 2
            ids1 = plsc.load_gather(idxb, [lane3 + 48 * s + 1]) * 2
            ids2 = plsc.load_gather(idxb, [lane3 + 48 * s + 2]) * 2
            x0 = plsc.load_gather(nodes_v, [ids0])
            y0 = plsc.load_gather(nodes_v, [ids0 + 1])
            x1 = plsc.load_gather(nodes_v, [ids1])
            y1 = plsc.load_gather(nodes_v, [ids1 + 1])
            x2 = plsc.load_gather(nodes_v, [ids2])
            y2 = plsc.load_gather(nodes_v, [ids2 + 1])
            e1x = x1 - x0
            e1y = y1 - y0
            e2x = x2 - x0
            e2y = y2 - y0
            detj = e1x * e2y - e1y * e2x
            ag = e1x * e1x + e1y * e1y
            bg = e1x * e2x + e1y * e2y
            cg = e2x * e2x + e2y * e2y
            sc = ws / detj
            wbuf[pl.ds(0, _L)] = detj * m00 + sc * (ag - 2.0 * bg + cg)
            wbuf[pl.ds(_L, _L)] = detj * m11 + sc * cg
            wbuf[pl.ds(2 * _L, _L)] = detj * m22 + sc * ag
            wbuf[pl.ds(3 * _L, _L)] = 2.0 * (detj * m01 - sc * (cg - bg))
            wbuf[pl.ds(4 * _L, _L)] = 2.0 * (detj * m02 - sc * (ag - bg))
            wbuf[pl.ds(5 * _L, _L)] = 2.0 * (detj * m12 - sc * bg)

            def el_body(l, a_in):
                r0 = 48 * s + l * 3
                va = rowsb[r0, pl.ds(0, _L)]
                vb = rowsb[r0 + 1, pl.ds(0, _L)]
                vc = rowsb[r0 + 2, pl.ds(0, _L)]
                d0 = va * va
                d1 = vb * vb
                d2 = vc * vc
                d3 = va * vb
                d4 = va * vc
                d5 = vb * vc
                for j in range(_L, 256, _L):
                    va = rowsb[r0, pl.ds(j, _L)]
                    vb = rowsb[r0 + 1, pl.ds(j, _L)]
                    vc = rowsb[r0 + 2, pl.ds(j, _L)]
                    d0 = d0 + va * va
                    d1 = d1 + vb * vb
                    d2 = d2 + vc * vc
                    d3 = d3 + va * vb
                    d4 = d4 + va * vc
                    d5 = d5 + vb * vc
                li = jnp.full((_L,), l, dtype=jnp.int32)
                w0 = plsc.load_gather(wbuf, [li])
                w1 = plsc.load_gather(wbuf, [li + _L])
                w2 = plsc.load_gather(wbuf, [li + 2 * _L])
                w3 = plsc.load_gather(wbuf, [li + 3 * _L])
                w4 = plsc.load_gather(wbuf, [li + 4 * _L])
                w5 = plsc.load_gather(wbuf, [li + 5 * _L])
                return (a_in + (w0 * d0 + w1 * d1) + (w2 * d2 + w3 * d3)
                        + (w4 * d4 + w5 * d5))

            acc = lax.fori_loop(0, _EG_INNER, el_body, acc)
        return acc

    issue(2 * u_start, idxb0, rows0, sem0)

    def unit_body(u, acc):
        issue(2 * u + 1, idxb1, rows1, sem1)
        drain(idxb0, rows0, sem0)
        acc = compute(idxb0, rows0, acc)
        issue(jnp.minimum(2 * u + 2, g_last), idxb0, rows0, sem0)
        drain(idxb1, rows1, sem1)
        acc = compute(idxb1, rows1, acc)
        return acc

    acc = lax.fori_loop(u_start, u_end, unit_body,
                        jnp.zeros((_L,), jnp.float32))
    drain(idxb0, rows0, sem0)
    accv[...] = acc
    pltpu.sync_copy(accv, out_hbm.at[wid])


_EG_INNER = 16


def _run_sc(nvals, nodes_xy, idx_flat, coefs):
    n_nodes = nodes_xy.shape[0] // 2
    mesh = plsc.VectorSubcoreMesh(core_axis_name="c", subcore_axis_name="s")
    f = pl.kernel(
        _sc_body,
        out_type=jax.ShapeDtypeStruct((_NW, _L), jnp.float32),
        mesh=mesh,
        scratch_types=[
            pltpu.VMEM((2 * n_nodes,), jnp.float32),
            pltpu.VMEM((_RPG,), jnp.int32),
            pltpu.VMEM((_RPG,), jnp.int32),
            pltpu.VMEM((_RPG, 256), jnp.float32),
            pltpu.VMEM((_RPG, 256), jnp.float32),
            pltpu.VMEM((6 * _L,), jnp.float32),
            pltpu.VMEM((_L,), jnp.float32),
            pltpu.VMEM((_L,), jnp.float32),
            pltpu.SemaphoreType.DMA,
            pltpu.SemaphoreType.DMA,
        ],
        compiler_params=pltpu.CompilerParams(needs_layout_passes=False),
    )
    return f(nvals, nodes_xy, idx_flat, coefs)


def kernel(nodal_values, nodes, elements, quad_points, quad_weights):
    qx = quad_points[:, 0]
    qy = quad_points[:, 1]
    shp = jnp.stack([1.0 - qx - qy, qx, qy], axis=1)
    m = jnp.einsum('q,qa,qb->ab', quad_weights, shp, shp)
    coefs = jnp.zeros((16,), jnp.float32)
    coefs = coefs.at[:7].set(jnp.stack(
        [m[0, 0], m[1, 1], m[2, 2], m[0, 1], m[0, 2], m[1, 2],
         jnp.sum(quad_weights)]))
    idx_flat = elements.reshape(-1).astype(jnp.int32)
    partials = _run_sc(nodal_values, nodes.reshape(-1), idx_flat, coefs)
    return jnp.sum(partials)
